# Initial kernel scaffold; baseline (speedup 1.0000x reference)
import functools

import jax
import jax.numpy as jnp
from jax import lax
from jax.experimental import pallas as pl
from jax.experimental.pallas import tpu as pltpu
from jax.experimental.pallas import tpu_sc as plsc

NUM_TOKENS = 16384 * 50
DIM = 64
NC, NS = 2, 16
NW = NC * NS
IDS_PER_GATHER = 128
ROWS_PER_WORKER = NUM_TOKENS // (NW * IDS_PER_GATHER)
CHUNK_ROWS = 8
NCHUNKS = ROWS_PER_WORKER // CHUNK_ROWS
CHUNK_IDS = CHUNK_ROWS * IDS_PER_GATHER

_mesh = plsc.VectorSubcoreMesh(core_axis_name="c", subcore_axis_name="s")


@functools.partial(
    pl.kernel,
    out_type=jax.ShapeDtypeStruct((NUM_TOKENS, DIM), jnp.float32),
    mesh=_mesh,
    scratch_types=[
        pltpu.VMEM((CHUNK_ROWS, IDS_PER_GATHER), jnp.int32),
        pltpu.VMEM((CHUNK_IDS, DIM), jnp.float32),
        pltpu.SemaphoreType.DMA,
    ],
)
def _emb_lookup(idx_hbm, table_hbm, out_hbm, idx_v, rows_v, sem):
    wid = lax.axis_index("s") * NC + lax.axis_index("c")
    row0 = wid * ROWS_PER_WORKER

    @pl.loop(0, NCHUNKS)
    def _chunk(i):
        rbase = row0 + i * CHUNK_ROWS
        pltpu.sync_copy(idx_hbm.at[pl.ds(rbase, CHUNK_ROWS)], idx_v)
        copies = [
            pltpu.async_copy(
                table_hbm.at[idx_v.at[j]],
                rows_v.at[pl.ds(j * IDS_PER_GATHER, IDS_PER_GATHER)],
                sem,
            )
            for j in range(CHUNK_ROWS)
        ]
        for cp in copies:
            cp.wait()
        pltpu.sync_copy(
            rows_v, out_hbm.at[pl.ds(rbase * IDS_PER_GATHER, CHUNK_IDS)]
        )


def kernel(token_ids, W):
    n, s = token_ids.shape
    idx = token_ids.reshape(NUM_TOKENS // IDS_PER_GATHER, IDS_PER_GATHER)
    idx = idx.astype(jnp.int32)
    out = _emb_lookup(idx, W)
    return out.reshape(n, s, DIM)


# speedup vs baseline: 1.8455x; 1.8455x over previous
import functools

import jax
import jax.numpy as jnp
from jax import lax
from jax.experimental import pallas as pl
from jax.experimental.pallas import tpu as pltpu
from jax.experimental.pallas import tpu_sc as plsc

NUM_TOKENS = 16384 * 50
DIM = 64
NC, NS = 2, 16
NW = NC * NS
IDS_PER_GATHER = 128
ROWS_PER_WORKER = NUM_TOKENS // (NW * IDS_PER_GATHER)
CHUNK_ROWS = 8
NCHUNKS = ROWS_PER_WORKER // CHUNK_ROWS
CHUNK_IDS = CHUNK_ROWS * IDS_PER_GATHER

_mesh = plsc.VectorSubcoreMesh(core_axis_name="c", subcore_axis_name="s")


@functools.partial(
    pl.kernel,
    out_type=jax.ShapeDtypeStruct((NUM_TOKENS, DIM), jnp.float32),
    mesh=_mesh,
    scratch_types=[
        pltpu.VMEM((CHUNK_ROWS, IDS_PER_GATHER), jnp.int32),
        pltpu.VMEM((CHUNK_IDS, DIM), jnp.float32),
        pltpu.SemaphoreType.DMA,
    ],
    compiler_params=pltpu.CompilerParams(use_tc_tiling_on_sc=False),
)
def _emb_lookup(idx_hbm, table_hbm, out_hbm, idx_v, rows_v, sem):
    wid = lax.axis_index("s") * NC + lax.axis_index("c")
    row0 = wid * ROWS_PER_WORKER

    @pl.loop(0, NCHUNKS)
    def _chunk(i):
        rbase = row0 + i * CHUNK_ROWS
        pltpu.sync_copy(idx_hbm.at[pl.ds(rbase, CHUNK_ROWS)], idx_v)
        copies = [
            pltpu.async_copy(
                table_hbm.at[idx_v.at[j]],
                rows_v.at[pl.ds(j * IDS_PER_GATHER, IDS_PER_GATHER)],
                sem,
            )
            for j in range(CHUNK_ROWS)
        ]
        for cp in copies:
            cp.wait()
        pltpu.sync_copy(
            rows_v, out_hbm.at[pl.ds(rbase * IDS_PER_GATHER, CHUNK_IDS)]
        )


def kernel(token_ids, W):
    n, s = token_ids.shape
    idx = token_ids.reshape(NUM_TOKENS // IDS_PER_GATHER, IDS_PER_GATHER)
    idx = idx.astype(jnp.int32)
    out = _emb_lookup(idx, W)
    return out.reshape(n, s, DIM)


# speedup vs baseline: 1.8737x; 1.0153x over previous
import functools

import jax
import jax.numpy as jnp
from jax import lax
from jax.experimental import pallas as pl
from jax.experimental.pallas import tpu as pltpu
from jax.experimental.pallas import tpu_sc as plsc

NUM_TOKENS = 16384 * 50
DIM = 64
NC, NS = 2, 16
NW = NC * NS
IPG = 128
ROWS_PER_WORKER = NUM_TOKENS // (NW * IPG)
C_ROWS = 5
CHUNK_IDS = C_ROWS * IPG
NCHUNKS = ROWS_PER_WORKER // C_ROWS
NPAIR = NCHUNKS // 2

_mesh = plsc.VectorSubcoreMesh(core_axis_name="c", subcore_axis_name="s")


@functools.partial(
    pl.kernel,
    out_type=jax.ShapeDtypeStruct((NUM_TOKENS, DIM), jnp.float32),
    mesh=_mesh,
    scratch_types=[
        pltpu.VMEM((ROWS_PER_WORKER, IPG), jnp.int32),
        pltpu.VMEM((2, CHUNK_IDS, DIM), jnp.float32),
        pltpu.SemaphoreType.DMA,
        pltpu.SemaphoreType.DMA,
        pltpu.SemaphoreType.DMA,
        pltpu.SemaphoreType.DMA,
    ],
    compiler_params=pltpu.CompilerParams(use_tc_tiling_on_sc=False),
)
def _emb_lookup(idx_hbm, table_hbm, out_hbm, idx_v, rows_v, gsem0, gsem1,
                ssem0, ssem1):
    wid = lax.axis_index("s") * NC + lax.axis_index("c")
    row0 = wid * ROWS_PER_WORKER
    out0 = row0 * IPG

    pltpu.sync_copy(idx_hbm.at[pl.ds(row0, ROWS_PER_WORKER)], idx_v)

    def fire_gathers(c, buf, sem):
        for j in range(C_ROWS):
            pltpu.async_copy(
                table_hbm.at[idx_v.at[c * C_ROWS + j]],
                rows_v.at[buf].at[pl.ds(j * IPG, IPG)],
                sem,
            )

    def wait_gathers(buf, sem):
        for j in range(C_ROWS):
            pltpu.make_async_copy(
                table_hbm.at[pl.ds(0, IPG)],
                rows_v.at[buf].at[pl.ds(j * IPG, IPG)],
                sem,
            ).wait()

    def fire_store(c, buf, sem):
        pltpu.async_copy(
            rows_v.at[buf], out_hbm.at[pl.ds(out0 + c * CHUNK_IDS, CHUNK_IDS)],
            sem,
        )

    def wait_store(buf, sem):
        pltpu.make_async_copy(
            rows_v.at[buf], out_hbm.at[pl.ds(0, CHUNK_IDS)], sem,
        ).wait()

    fire_gathers(0, 0, gsem0)

    @pl.loop(0, NPAIR)
    def _pair(k):
        i = 2 * k

        @pl.when(k > 0)
        def _():
            wait_store(1, ssem1)
        fire_gathers(i + 1, 1, gsem1)

        wait_gathers(0, gsem0)
        fire_store(i, 0, ssem0)

        @pl.when(k < NPAIR - 1)
        def _():
            wait_store(0, ssem0)
            fire_gathers(i + 2, 0, gsem0)

        wait_gathers(1, gsem1)
        fire_store(i + 1, 1, ssem1)

    wait_store(0, ssem0)
    wait_store(1, ssem1)


def kernel(token_ids, W):
    n, s = token_ids.shape
    idx = token_ids.reshape(NUM_TOKENS // IPG, IPG).astype(jnp.int32)
    out = _emb_lookup(idx, W)
    return out.reshape(n, s, DIM)
